# baseline (speedup 1.0000x reference)
import functools

import jax
import jax.numpy as jnp
from jax import lax
from jax.experimental import pallas as pl
from jax.experimental.pallas import tpu as pltpu

B, N, D = 32, 4096, 1024
TOP_K = 25
NBLK = 4
BLK = N // NBLK
NQ = 4
QW = N // NQ
CAND = 32


def _tc_body(gate_ref, q_ref, k_ref, out_ref, s_ref, att_ref, rs_ref):
    step = pl.program_id(0)

    @pl.when(step == 0)
    def _init():
        rs_ref[...] = jnp.zeros((B, 128), jnp.float32)

    q = q_ref[...]
    k = k_ref[...]
    qh = q / jnp.maximum(
        jnp.sqrt(jnp.sum(q * q, axis=1, keepdims=True)), 1e-12
    )
    kh = k / jnp.maximum(
        jnp.sqrt(jnp.sum(k * k, axis=1, keepdims=True)), 1e-12
    )
    qn = jnp.maximum(jnp.sqrt(jnp.sum(qh * qh, axis=1, keepdims=True)), 1e-8)
    kn = jnp.maximum(jnp.sqrt(jnp.sum(kh * kh, axis=1, keepdims=True)), 1e-8)
    dot = jax.lax.dot_general(
        qh, kh, (((1,), (1,)), ((), ())), preferred_element_type=jnp.float32
    )
    s = jnp.clip(dot / (qn * kn.reshape(1, BLK)), 0.0, 1.0)
    s_ref[:, pl.ds(step * BLK, BLK)] = s

    e = jnp.exp(s)
    att_ref[:, pl.ds(step * BLK, BLK)] = e
    rs_ref[:, 0:1] += jnp.sum(e, axis=1, keepdims=True)

    @pl.when(step == NBLK - 1)
    def _finish():
        iota_q = lax.broadcasted_iota(jnp.int32, (B, QW), 1)
        lanec = lax.broadcasted_iota(jnp.int32, (B, CAND), 1)
        cvs = []
        cis = []
        for qtr in range(NQ):
            w = s_ref[:, qtr * QW:(qtr + 1) * QW]
            cv = jnp.full((B, CAND), -1.0, dtype=jnp.float32)
            ci = jnp.zeros((B, CAND), dtype=jnp.int32)
            for t in range(TOP_K):
                mx = jnp.max(w, axis=1, keepdims=True)
                li = jnp.min(
                    jnp.where(w == mx, iota_q, QW), axis=1, keepdims=True
                )
                cv = jnp.where(lanec == t, mx, cv)
                ci = jnp.where(lanec == t, li + qtr * QW, ci)
                w = jnp.where(iota_q == li, -1.0, w)
            cvs.append(cv)
            cis.append(ci)
        cw = jnp.concatenate(cvs, axis=1)
        cidx = jnp.concatenate(cis, axis=1)

        niota = lax.broadcasted_iota(jnp.int32, (B, N), 1)
        union = jnp.zeros((1, N), dtype=jnp.float32)
        selfset = jnp.zeros((1, N), dtype=jnp.float32)
        for t in range(TOP_K):
            mx = jnp.max(cw, axis=1, keepdims=True)
            gi = jnp.min(jnp.where(cw == mx, cidx, N), axis=1, keepdims=True)
            hit = jnp.max(
                (niota == gi).astype(jnp.float32), axis=0, keepdims=True
            )
            union = jnp.maximum(union, hit)
            if t == 0:
                selfset = hit
            cw = jnp.where((cw == mx) & (cidx == gi), -1.0, cw)

        mask = union * (1.0 - selfset)
        inv = gate_ref[0] / rs_ref[:, 0:1]
        out_ref[...] = att_ref[...] * inv * mask


def _tc_call(gate, q1, k1):
    return pl.pallas_call(
        _tc_body,
        grid=(NBLK,),
        in_specs=[
            pl.BlockSpec(memory_space=pltpu.SMEM),
            pl.BlockSpec((B, D), lambda i: (0, 0)),
            pl.BlockSpec((BLK, D), lambda i: (i, 0)),
        ],
        out_specs=pl.BlockSpec((B, N), lambda i: (0, 0)),
        out_shape=jax.ShapeDtypeStruct((B, N), jnp.float32),
        scratch_shapes=[
            pltpu.VMEM((B, N), jnp.float32),
            pltpu.VMEM((B, N), jnp.float32),
            pltpu.VMEM((B, 128), jnp.float32),
        ],
    )(gate, q1, k1)


@jax.jit
def kernel(q1, k1, q2, k2, q3, k3, temp_vid, temp_aud, temp_ocr):
    del q2, k2, q3, k3, temp_aud, temp_ocr
    gate = jax.nn.sigmoid(temp_vid) * 2.0
    return _tc_call(gate, q1, k1)


# speedup vs baseline: 1.2172x; 1.2172x over previous
import functools

import jax
import jax.numpy as jnp
from jax import lax
from jax.experimental import pallas as pl
from jax.experimental.pallas import tpu as pltpu

B, N, D = 32, 4096, 1024
TOP_K = 25
NBLK = 4
BLK = N // NBLK
NQ = 4
QW = N // NQ
CAND = 32


def _tc_body(gate_ref, q_ref, k_ref, out_ref, s_ref, att_ref, rs_ref):
    step = pl.program_id(0)

    @pl.when(step == 0)
    def _init():
        rs_ref[...] = jnp.zeros((B, 128), jnp.float32)

    q = q_ref[...]
    k = k_ref[...]
    qh = q / jnp.maximum(
        jnp.sqrt(jnp.sum(q * q, axis=1, keepdims=True)), 1e-12
    )
    kh = k / jnp.maximum(
        jnp.sqrt(jnp.sum(k * k, axis=1, keepdims=True)), 1e-12
    )
    qn = jnp.maximum(jnp.sqrt(jnp.sum(qh * qh, axis=1, keepdims=True)), 1e-8)
    kn = jnp.maximum(jnp.sqrt(jnp.sum(kh * kh, axis=1, keepdims=True)), 1e-8)
    dot = jax.lax.dot_general(
        qh, kh, (((1,), (1,)), ((), ())), preferred_element_type=jnp.float32
    )
    s = jnp.clip(dot / (qn * kn.reshape(1, BLK)), 0.0, 1.0)
    s_ref[:, pl.ds(step * BLK, BLK)] = s

    e = jnp.exp(s)
    att_ref[:, pl.ds(step * BLK, BLK)] = e
    rs_ref[:, 0:1] += jnp.sum(e, axis=1, keepdims=True)

    @pl.when(step == NBLK - 1)
    def _finish():
        w = s_ref[...]
        niota = lax.broadcasted_iota(jnp.int32, (B, N), 1)
        lane = lax.broadcasted_iota(jnp.int32, (B, 128), 1)
        union = jnp.zeros((1, N), dtype=jnp.float32)
        selfset = jnp.zeros((1, N), dtype=jnp.float32)
        ngrp = N // 128
        for t in range(TOP_K):
            accv = w[:, 0:128]
            accg = jnp.zeros((B, 128), dtype=jnp.int32)
            for g in range(1, ngrp):
                wg = w[:, g * 128:(g + 1) * 128]
                m = wg > accv
                accv = jnp.where(m, wg, accv)
                accg = jnp.where(m, g, accg)
            mx = jnp.max(accv, axis=1, keepdims=True)
            nstar = jnp.min(
                jnp.where(accv == mx, accg * 128 + lane, N),
                axis=1,
                keepdims=True,
            )
            sel = niota == nstar
            hit = jnp.max(sel.astype(jnp.float32), axis=0, keepdims=True)
            union = jnp.maximum(union, hit)
            if t == 0:
                selfset = hit
            w = jnp.where(sel, -1.0, w)

        mask = union * (1.0 - selfset)
        inv = gate_ref[0] / rs_ref[:, 0:1]
        out_ref[...] = att_ref[...] * inv * mask


def _tc_call(gate, q1, k1):
    return pl.pallas_call(
        _tc_body,
        grid=(NBLK,),
        in_specs=[
            pl.BlockSpec(memory_space=pltpu.SMEM),
            pl.BlockSpec((B, D), lambda i: (0, 0)),
            pl.BlockSpec((BLK, D), lambda i: (i, 0)),
        ],
        out_specs=pl.BlockSpec((B, N), lambda i: (0, 0)),
        out_shape=jax.ShapeDtypeStruct((B, N), jnp.float32),
        scratch_shapes=[
            pltpu.VMEM((B, N), jnp.float32),
            pltpu.VMEM((B, N), jnp.float32),
            pltpu.VMEM((B, 128), jnp.float32),
        ],
    )(gate, q1, k1)


@jax.jit
def kernel(q1, k1, q2, k2, q3, k3, temp_vid, temp_aud, temp_ocr):
    del q2, k2, q3, k3, temp_aud, temp_ocr
    gate = jax.nn.sigmoid(temp_vid) * 2.0
    return _tc_call(gate, q1, k1)
